# baseline (speedup 1.0000x reference)
import functools
import jax
import jax.numpy as jnp
from jax import lax
from jax.experimental import pallas as pl
from jax.experimental.pallas import tpu as pltpu
from jax.experimental.pallas import tpu_sc as plsc

_NC, _NS, _L = 2, 16, 16
_NW = _NC * _NS
_ROWS, _COLS = 16384, 200
_RPW = _ROWS // _NW
_CH = 128
_K = _RPW // _CH

_COL_OFFS = list(range(0, _COLS - _L + 1, _L))
if _COL_OFFS[-1] != _COLS - _L:
    _COL_OFFS.append(_COLS - _L)


def _sc_body(x_hbm, v_hbm, rv_hbm, out_hbm, buf0, buf1, vv, rvv,
             in_sem0, in_sem1, out_sem0, out_sem1):
    bufs = (buf0, buf1)
    in_sems = (in_sem0, in_sem1)
    out_sems = (out_sem0, out_sem1)
    wid = lax.axis_index("s") * _NC + lax.axis_index("c")
    base = wid * _RPW
    pltpu.sync_copy(v_hbm, vv)
    pltpu.sync_copy(rv_hbm, rvv)
    v = vv[...]
    rv = rvv[...]

    in_handles = [None] * _K
    out_handles = [None] * _K

    def start_in(k):
        b = k % 2
        in_handles[k] = pltpu.async_copy(
            x_hbm.at[pl.ds(base + k * _CH, _CH)], bufs[b], in_sems[b])

    def start_out(k):
        b = k % 2
        out_handles[k] = pltpu.async_copy(
            bufs[b], out_hbm.at[pl.ds(base + k * _CH, _CH)], out_sems[b])

    start_in(0)
    for k in range(_K):
        b = k % 2
        if k + 1 < _K:
            if k >= 1:
                out_handles[k - 1].wait()
            start_in(k + 1)
        in_handles[k].wait()
        buf = bufs[b]

        def row_body(r_i, carry):
            for c in _COL_OFFS:
                x = buf[r_i, pl.ds(c, _L)]
                q = (x.astype(jnp.float32) * rv).astype(jnp.int32)
                r = x - q * v
                r = jnp.where(r < 0, r + v, r)
                r = jnp.where(r >= v, r - v, r)
                buf[r_i, pl.ds(c, _L)] = r
            return carry

        lax.fori_loop(0, _CH, row_body, 0)
        start_out(k)
    out_handles[_K - 2].wait()
    out_handles[_K - 1].wait()


def kernel(indices, feature_idx, vocab_sizes):
    vocab = jax.lax.dynamic_index_in_dim(vocab_sizes, feature_idx, keepdims=False)
    vv = jnp.full((_L,), vocab, dtype=jnp.int32)
    rvv = jnp.full((_L,), 1.0 / vocab.astype(jnp.float32), dtype=jnp.float32)
    mesh = plsc.VectorSubcoreMesh(
        core_axis_name="c", subcore_axis_name="s", num_cores=_NC, num_subcores=_NS
    )
    f = functools.partial(
        pl.kernel,
        out_type=jax.ShapeDtypeStruct((_ROWS, _COLS), indices.dtype),
        mesh=mesh,
        scratch_types=[
            pltpu.VMEM((_CH, _COLS), jnp.int32),
            pltpu.VMEM((_CH, _COLS), jnp.int32),
            pltpu.VMEM((_L,), jnp.int32),
            pltpu.VMEM((_L,), jnp.float32),
            pltpu.SemaphoreType.DMA,
            pltpu.SemaphoreType.DMA,
            pltpu.SemaphoreType.DMA,
            pltpu.SemaphoreType.DMA,
        ],
    )(_sc_body)
    return f(indices, vv, rvv)


# speedup vs baseline: 1.9337x; 1.9337x over previous
import functools
import jax
import jax.numpy as jnp
from jax.experimental import pallas as pl
from jax.experimental.pallas import tpu as pltpu

_ROWS, _COLS = 16384, 200
_CHR = 1024
_K = _ROWS // _CHR
_NBUF = 4


def _body(v_ref, rv_ref, x_hbm, o_hbm, *rest):
    ibufs = rest[:_NBUF]
    obufs = rest[_NBUF:2 * _NBUF]
    in_sems = rest[2 * _NBUF:3 * _NBUF]
    out_sems = rest[3 * _NBUF:4 * _NBUF]
    v = v_ref[0]
    rv = rv_ref[0]

    in_h = [None] * (_K + _NBUF)
    out_h = [None] * _K

    def start_in(k):
        b = k % _NBUF
        in_h[k] = pltpu.make_async_copy(
            x_hbm.at[pl.ds(k * _CHR, _CHR)], ibufs[b], in_sems[b])
        in_h[k].start()

    def start_out(k):
        b = k % _NBUF
        out_h[k] = pltpu.make_async_copy(
            obufs[b], o_hbm.at[pl.ds(k * _CHR, _CHR)], out_sems[b])
        out_h[k].start()

    for k in range(min(_NBUF, _K)):
        start_in(k)
    for k in range(_K):
        b = k % _NBUF
        if k >= _NBUF:
            out_h[k - _NBUF].wait()
        in_h[k].wait()
        x = ibufs[b][...]
        q = jnp.floor(x.astype(jnp.float32) * rv).astype(jnp.int32)
        r = x - q * v
        r = jnp.where(r < 0, r + v, r)
        r = jnp.where(r >= v, r - v, r)
        obufs[b][...] = r
        if k + _NBUF < _K:
            start_in(k + _NBUF)
        start_out(k)
    for k in range(max(_K - _NBUF, 0), _K):
        out_h[k].wait()


def kernel(indices, feature_idx, vocab_sizes):
    vocab = jax.lax.dynamic_index_in_dim(vocab_sizes, feature_idx, keepdims=True)
    recip = 1.0 / vocab.astype(jnp.float32)
    scratch = (
        [pltpu.VMEM((_CHR, _COLS), jnp.int32)] * (2 * _NBUF)
        + [pltpu.SemaphoreType.DMA] * (2 * _NBUF)
    )
    return pl.pallas_call(
        _body,
        in_specs=[
            pl.BlockSpec(memory_space=pltpu.SMEM),
            pl.BlockSpec(memory_space=pltpu.SMEM),
            pl.BlockSpec(memory_space=pl.ANY),
        ],
        out_specs=pl.BlockSpec(memory_space=pl.ANY),
        out_shape=jax.ShapeDtypeStruct((_ROWS, _COLS), indices.dtype),
        scratch_shapes=scratch,
    )(vocab, recip, indices)
